# baseline (speedup 1.0000x reference)
import functools

import jax
import jax.numpy as jnp
from jax import lax
from jax.experimental import pallas as pl
from jax.experimental.pallas import tpu as pltpu
from jax.experimental.pallas import tpu_sc as plsc

N = 10000
NP = 10016
NH = 5000
E = 160000
G = 64
NC, NS = 2, 16
NW = NC * NS
CAPW = 6144
CHK = 32
CHN = CAPW // CHK
CAP = CAPW * NS
EP2 = 2 * CAP
ACCR = 5008
FLR = ACCR // NS
MB = 1000


def _mm_tiled(x_t, w_t):
    tin, m, _ = x_t.shape
    nout = w_t.shape[2]
    tout = nout // 256

    def body(x_ref, w_ref, o_ref):
        @pl.when(pl.program_id(2) == 0)
        def _():
            o_ref[...] = jnp.zeros_like(o_ref)
        o_ref[...] += jnp.dot(x_ref[0], w_ref[0],
                              preferred_element_type=jnp.float32)[None]

    return pl.pallas_call(
        body,
        grid=(m // MB, tout, tin),
        in_specs=[
            pl.BlockSpec((1, MB, 256), lambda i, j, k: (k, i, 0)),
            pl.BlockSpec((1, 256, 256), lambda i, j, k: (k, 0, j)),
        ],
        out_specs=pl.BlockSpec((1, MB, 256), lambda i, j, k: (j, i, 0)),
        out_shape=jax.ShapeDtypeStruct((tout, m, 256), jnp.float32),
        compiler_params=pltpu.CompilerParams(
            dimension_semantics=("parallel", "parallel", "arbitrary")),
    )(x_t, w_t)


def _colmax(a):
    m = a.shape[0]

    def body(a_ref, o_ref):
        @pl.when(pl.program_id(0) == 0)
        def _():
            o_ref[...] = jnp.full_like(o_ref, -jnp.inf)
        mx = jnp.max(a_ref[...], axis=0, keepdims=True)
        o_ref[...] = jnp.maximum(o_ref[...], jnp.broadcast_to(mx, o_ref.shape))

    return pl.pallas_call(
        body,
        grid=(m // MB,),
        in_specs=[pl.BlockSpec((MB, 256), lambda i: (i, 0))],
        out_specs=pl.BlockSpec((8, 256), lambda i: (0, 0)),
        out_shape=jax.ShapeDtypeStruct((8, 256), jnp.float32),
        compiler_params=pltpu.CompilerParams(
            dimension_semantics=("arbitrary",)),
    )(a)


def _den_sum(pden):
    heads = pden.shape[0]

    def body(p_ref, o_ref):
        o_ref[...] = jnp.sum(p_ref[...], axis=1)

    return pl.pallas_call(
        body,
        out_shape=jax.ShapeDtypeStruct((heads, NP), jnp.float32),
    )(pden)


def _finish(acc2, bias_t):
    t = acc2.shape[0]

    def body(a_ref, b_ref, o_ref):
        s = a_ref[0]
        b = jnp.broadcast_to(b_ref[0][0:1, :], s.shape)
        o_ref[...] = jnp.maximum(s + b, 0.0)[None]

    return pl.pallas_call(
        body,
        grid=(N // MB, t),
        in_specs=[
            pl.BlockSpec((1, MB, 256), lambda i, j: (j, i, 0)),
            pl.BlockSpec((1, 8, 256), lambda i, j: (j, 0, 0)),
        ],
        out_specs=pl.BlockSpec((1, MB, 256), lambda i, j: (j, i, 0)),
        out_shape=jax.ShapeDtypeStruct((t, N, 256), jnp.float32),
        compiler_params=pltpu.CompilerParams(
            dimension_semantics=("parallel", "parallel")),
    )(acc2, bias_t)


def _pool_gru(h3_t, p, w_ih1t, b_ih1, b_hh1, w_ih2t, b_ih2, b_hh2, wot, bo):
    def body(h_ref, p_ref, wi1_ref, bi1_ref, bh1_ref, wi2_ref, bi2_ref,
             bh2_ref, wo_ref, bo_ref, o_ref):
        pm = p_ref[...]
        dn = (((0,), (0,)), ((), ()))
        parts = [lax.dot_general(pm, h_ref[tt], dn,
                                 preferred_element_type=jnp.float32)
                 for tt in range(2)]
        ge = jnp.concatenate(parts, axis=1)
        cnt = lax.dot_general(pm, jnp.ones((N, 8), jnp.float32), dn,
                              preferred_element_type=jnp.float32)[:, 0:1]
        ge = ge / jnp.maximum(cnt, 1.0)

        gi1 = jnp.dot(ge, wi1_ref[...], preferred_element_type=jnp.float32)
        gi1 = gi1 + jnp.broadcast_to(bi1_ref[...], gi1.shape)
        bh1 = jnp.broadcast_to(bh1_ref[...], gi1.shape)
        r1 = jax.nn.sigmoid(gi1[:, 0:256] + bh1[:, 0:256])
        z1 = jax.nn.sigmoid(gi1[:, 256:512] + bh1[:, 256:512])
        n1 = jnp.tanh(gi1[:, 512:768] + r1 * bh1[:, 512:768])
        h1 = (1.0 - z1) * n1

        gi2 = jnp.dot(h1, wi2_ref[...], preferred_element_type=jnp.float32)
        gi2 = gi2 + jnp.broadcast_to(bi2_ref[...], gi2.shape)
        bh2 = jnp.broadcast_to(bh2_ref[...], gi2.shape)
        r2 = jax.nn.sigmoid(gi2[:, 0:256] + bh2[:, 0:256])
        z2 = jax.nn.sigmoid(gi2[:, 256:512] + bh2[:, 256:512])
        n2 = jnp.tanh(gi2[:, 512:768] + r2 * bh2[:, 512:768])
        h2 = (1.0 - z2) * n2

        out = jnp.dot(h2, wo_ref[...], preferred_element_type=jnp.float32)
        o_ref[...] = out + jnp.broadcast_to(bo_ref[...], out.shape)

    return pl.pallas_call(
        body,
        out_shape=jax.ShapeDtypeStruct((G, 512), jnp.float32),
    )(h3_t, p, w_ih1t, b_ih1, b_hh1, w_ih2t, b_ih2, b_hh2, wot, bo)


def _sc_mesh():
    return plsc.VectorSubcoreMesh(core_axis_name="c", subcore_axis_name="s")


_SC_PARAMS = pltpu.CompilerParams(needs_layout_passes=False,
                                  use_tc_tiling_on_sc=False)


def _edge_weights(heads, as_t, ad_t, ei4, cvec):
    @functools.partial(
        pl.kernel,
        out_type=(jax.ShapeDtypeStruct((heads, NW, CHN, CHK), jnp.float32),
                  jax.ShapeDtypeStruct((heads, NW, NP), jnp.float32)),
        mesh=_sc_mesh(),
        compiler_params=_SC_PARAMS,
        scratch_types=[
            pltpu.VMEM((CHN, CHK), jnp.int32),
            pltpu.VMEM((CHN, CHK), jnp.int32),
            pltpu.VMEM((NP,), jnp.float32),
            pltpu.VMEM((NP,), jnp.float32),
            pltpu.VMEM((NP,), jnp.float32),
            pltpu.VMEM((CHN, CHK), jnp.float32),
            pltpu.VMEM((16,), jnp.float32),
        ],
    )
    def ek(as_hbm, ad_hbm, ei_hbm, c_hbm, w_out, pden_out,
           src_v, dst_v, as_v, ad_v, den_v, w_v, c_v):
        cc = lax.axis_index("c")
        ss = lax.axis_index("s")
        wid = cc * NS + ss
        pltpu.sync_copy(ei_hbm.at[0, wid], src_v)
        pltpu.sync_copy(ei_hbm.at[1, wid], dst_v)
        pltpu.sync_copy(c_hbm, c_v)
        cv = c_v[...]

        def head_body(h, _):
            pltpu.sync_copy(as_hbm.at[h], as_v)
            pltpu.sync_copy(ad_hbm.at[h], ad_v)

            def zero(i, _):
                den_v[pl.ds(i * 16, 16)] = jnp.zeros((16,), jnp.float32)
                return 0
            lax.fori_loop(0, NP // 16, zero, 0)

            def chunk(j, _):
                for q in range(CHK // 16):
                    s16 = src_v[j, pl.ds(q * 16, 16)]
                    d16 = dst_v[j, pl.ds(q * 16, 16)]
                    av = plsc.load_gather(as_v, [s16])
                    bv = plsc.load_gather(ad_v, [d16])
                    e = av + bv
                    e = jnp.where(e > 0, e, 0.2 * e)
                    wv = jnp.exp(e - cv)
                    w_v[j, pl.ds(q * 16, 16)] = wv
                    plsc.addupdate_scatter(den_v, [d16], wv)
                return 0
            lax.fori_loop(0, CHN, chunk, 0)

            pltpu.sync_copy(w_v, w_out.at[h, wid])
            pltpu.sync_copy(den_v, pden_out.at[h, wid])
            return 0
        lax.fori_loop(0, heads, head_body, 0)

    return ek(as_t, ad_t, ei4, cvec)


def _alphaize(heads, w4, den, ei4):
    @functools.partial(
        pl.kernel,
        out_type=jax.ShapeDtypeStruct((heads, NW, CHN, CHK), jnp.float32),
        mesh=_sc_mesh(),
        compiler_params=_SC_PARAMS,
        scratch_types=[
            pltpu.VMEM((CHN, CHK), jnp.int32),
            pltpu.VMEM((CHN, CHK), jnp.float32),
            pltpu.VMEM((NP,), jnp.float32),
        ],
    )
    def alk(w_hbm, den_hbm, ei_hbm, a_out, dst_v, w_v, den_v):
        cc = lax.axis_index("c")
        ss = lax.axis_index("s")
        wid = cc * NS + ss
        pltpu.sync_copy(ei_hbm.at[1, wid], dst_v)

        def head_body(h, _):
            pltpu.sync_copy(den_hbm.at[h], den_v)
            pltpu.sync_copy(w_hbm.at[h, wid], w_v)

            def chunk(j, _):
                for q in range(CHK // 16):
                    d16 = dst_v[j, pl.ds(q * 16, 16)]
                    dn = plsc.load_gather(den_v, [d16])
                    w_v[j, pl.ds(q * 16, 16)] = (
                        w_v[j, pl.ds(q * 16, 16)] / (dn + 1e-16))
                return 0
            lax.fori_loop(0, CHN, chunk, 0)

            pltpu.sync_copy(w_v, a_out.at[h, wid])
            return 0
        lax.fori_loop(0, heads, head_body, 0)

    return alk(w4, den, ei4)


def _aggregate(dim, tiles, h2d, alpha4, ei4, zer):
    @functools.partial(
        pl.kernel,
        out_type=jax.ShapeDtypeStruct((tiles, NC, ACCR, 256), jnp.float32),
        mesh=_sc_mesh(),
        compiler_params=_SC_PARAMS,
        scratch_types=[
            pltpu.VMEM((CHN, CHK), jnp.int32),
            pltpu.VMEM((CHN, CHK), jnp.int32),
            pltpu.VMEM((2, CHN, CHK), jnp.float32),
            pltpu.VMEM((CHK, 256), jnp.float32),
            pltpu.VMEM((CHK, 256), jnp.float32),
            pltpu.VMEM_SHARED((ACCR, 256), jnp.float32),
            pltpu.SemaphoreType.DMA,
            pltpu.SemaphoreType.DMA,
        ],
    )
    def ak(h_hbm, a_hbm, ei_hbm, z_hbm, acc_out,
           src_v, dst_v, a_v, rows_0, rows_1, acc_sp, sem_0, sem_1):
        rows_bufs = (rows_0, rows_1)
        sems = (sem_0, sem_1)
        cc = lax.axis_index("c")
        ss = lax.axis_index("s")
        wid = cc * NS + ss
        pltpu.sync_copy(ei_hbm.at[0, wid], src_v)
        pltpu.sync_copy(ei_hbm.at[1, wid], dst_v)

        base = cc * NH

        lane = lax.iota(jnp.int32, 16)

        def localize(j, _):
            for q in range(CHK // 16):
                d16 = dst_v[j, pl.ds(q * 16, 16)]
                loc = jnp.minimum(jnp.maximum(d16 - base, 0), NH)
                sp = j * CHK + q * 16 + lane
                sp = jnp.where(sp >= NH, sp - NH, sp)
                d16 = jnp.where(d16 == N, sp, loc)
                dst_v[j, pl.ds(q * 16, 16)] = d16
            return 0
        lax.fori_loop(0, CHN, localize, 0)

        def scale_scatter(j, rows):
            @plsc.parallel_loop(0, CHK, unroll=8)
            def _(i):
                wv0 = plsc.load_gather(
                    a_v, [jnp.full((16,), 0, jnp.int32),
                          jnp.full((16,), j, jnp.int32),
                          jnp.full((16,), i, jnp.int32)])
                wv1 = plsc.load_gather(
                    a_v, [jnp.full((16,), 1, jnp.int32),
                          jnp.full((16,), j, jnp.int32),
                          jnp.full((16,), i, jnp.int32)])
                for q in range(8):
                    rows[i, pl.ds(q * 16, 16)] = (
                        rows[i, pl.ds(q * 16, 16)] * wv0)
                for q in range(8, 16):
                    rows[i, pl.ds(q * 16, 16)] = (
                        rows[i, pl.ds(q * 16, 16)] * wv1)

            pltpu.sync_copy(rows, acc_sp.at[dst_v.at[j]], add=True)
            for q in range(CHK // 16):
                src_v[j, pl.ds(q * 16, 16)] = (
                    src_v[j, pl.ds(q * 16, 16)] + N)

        def tile_body(t, _):
            hh0 = (t * 256) // dim
            hh1 = (t * 256 + 128) // dim
            pltpu.sync_copy(a_hbm.at[hh0, wid], a_v.at[0])
            pltpu.sync_copy(a_hbm.at[hh1, wid], a_v.at[1])

            pltpu.sync_copy(z_hbm, acc_sp.at[pl.ds(ss * FLR, FLR)])
            plsc.subcore_barrier()

            for b in range(2):
                pltpu.async_copy(h_hbm.at[src_v.at[b]],
                                 rows_bufs[b], sems[b])

            def duo(jj, _):
                j0 = jj * 2
                for b in range(2):
                    j = j0 + b
                    pltpu.make_async_copy(
                        h_hbm.at[src_v.at[j]], rows_bufs[b], sems[b]).wait()
                    scale_scatter(j, rows_bufs[b])

                    @pl.when(j + 2 < CHN)
                    def _():
                        pltpu.async_copy(h_hbm.at[src_v.at[j + 2]],
                                         rows_bufs[b], sems[b])
                return 0
            lax.fori_loop(0, CHN // 2, duo, 0)
            plsc.subcore_barrier()

            sl = pl.ds(ss * FLR, FLR)
            pltpu.sync_copy(acc_sp.at[sl], acc_out.at[t, cc, sl])
            plsc.subcore_barrier()
            return 0
        lax.fori_loop(0, tiles, tile_body, 0)

    return ak(h2d, alpha4, ei4, zer)


def _gat_layer(x_t, w_t, a_src, a_dst, bias, heads, dim, ei4, zer):
    hp_t = _mm_tiled(x_t, w_t)
    t = hp_t.shape[0]
    k = heads * dim

    eye = jnp.eye(heads, dtype=jnp.float32)
    a_s = (eye[:, None, :] * a_src[:, :, None]).reshape(k, heads)
    a_d = (eye[:, None, :] * a_dst[:, :, None]).reshape(k, heads)
    a_cat = jnp.concatenate([a_s, a_d], axis=1)
    a_cat = jnp.pad(a_cat, ((0, 0), (0, 256 - 2 * heads)))
    a_cat = a_cat.reshape(k // 256, 256, 256)

    al = _mm_tiled(hp_t, a_cat)[0]
    m8 = _colmax(al)
    cb = jnp.maximum(
        jnp.max(m8[:, :heads]) + jnp.max(m8[:, heads:2 * heads]), 0.0)
    cvec = jnp.full((16,), cb, jnp.float32)

    al_tr = al.T
    as_t = jnp.pad(al_tr[:heads], ((0, 0), (0, NP - N)))
    ad_t = jnp.pad(al_tr[heads:2 * heads], ((0, 0), (0, NP - N)),
                   constant_values=-1e30)

    w_e, pden = _edge_weights(heads, as_t, ad_t, ei4, cvec)
    den = _den_sum(pden)
    alpha4 = _alphaize(heads, w_e, den, ei4)
    acc = _aggregate(dim, t, hp_t.reshape(t * N, 256), alpha4, ei4, zer)
    acc2 = acc[:, :, :NH, :].reshape(t, N, 256)
    bias_t = jnp.broadcast_to(bias.reshape(t, 1, 256), (t, 8, 256))
    return _finish(acc2, bias_t)


def kernel(x, edge_index, batch_idx, W1, a_src1, a_dst1, b1, W2, a_src2,
           a_dst2, b2, W3, a_src3, a_dst3, b3, W_ih1, W_hh1, b_ih1, b_hh1,
           W_ih2, W_hh2, b_ih2, b_hh2, Wo, bo):
    ei = edge_index.astype(jnp.int32)
    src, dst = ei[0], ei[1]
    m = (dst < NH).astype(jnp.int32)
    c0 = jnp.cumsum(m)
    c1 = jnp.cumsum(1 - m)
    pos = jnp.where(m == 1, c0 - 1, CAP + c1 - 1)
    bsrc = jnp.zeros((EP2,), jnp.int32).at[pos].set(src)
    bdst = jnp.full((EP2,), N, jnp.int32).at[pos].set(dst)
    ei4 = jnp.stack([bsrc, bdst]).reshape(2, NW, CHN, CHK)
    zer = jnp.zeros((FLR, 256), jnp.float32)

    x_t = jnp.pad(x, ((0, 0), (0, 256 - 47)))[None]
    w1_t = jnp.pad(W1, ((0, 256 - 47), (0, 0)))[None]
    h1_t = _gat_layer(x_t, w1_t, a_src1, a_dst1, b1, 8, 128, ei4, zer)
    h2_t = _gat_layer(h1_t, W2.reshape(4, 256, 2048), a_src2, a_dst2, b2,
                      8, 256, ei4, zer)
    h3_t = _gat_layer(h2_t, W3.reshape(8, 256, 512), a_src3, a_dst3, b3,
                      1, 512, ei4, zer)

    p = (batch_idx[:, None] == jnp.arange(G, dtype=batch_idx.dtype)[None, :])
    p = p.astype(jnp.float32)
    return _pool_gru(h3_t, p, W_ih1.T, b_ih1[None], b_hh1[None], W_ih2.T,
                     b_ih2[None], b_hh2[None], Wo.T, bo[None])


# speedup vs baseline: 1.0308x; 1.0308x over previous
import functools

import jax
import jax.numpy as jnp
from jax import lax
from jax.experimental import pallas as pl
from jax.experimental.pallas import tpu as pltpu
from jax.experimental.pallas import tpu_sc as plsc

N = 10000
NP = 10016
NH = 5000
E = 160000
G = 64
NC, NS = 2, 16
NW = NC * NS
CAPW = 6144
CHK = 32
CHN = CAPW // CHK
CAP = CAPW * NS
EP2 = 2 * CAP
ACCR = 5008
FLR = ACCR // NS
MB = 1000


def _mm_tiled(x_t, w_t):
    tin, m, _ = x_t.shape
    nout = w_t.shape[2]
    tout = nout // 256

    def body(x_ref, w_ref, o_ref):
        @pl.when(pl.program_id(2) == 0)
        def _():
            o_ref[...] = jnp.zeros_like(o_ref)
        o_ref[...] += jnp.dot(x_ref[0], w_ref[0],
                              preferred_element_type=jnp.float32)[None]

    return pl.pallas_call(
        body,
        grid=(m // MB, tout, tin),
        in_specs=[
            pl.BlockSpec((1, MB, 256), lambda i, j, k: (k, i, 0)),
            pl.BlockSpec((1, 256, 256), lambda i, j, k: (k, 0, j)),
        ],
        out_specs=pl.BlockSpec((1, MB, 256), lambda i, j, k: (j, i, 0)),
        out_shape=jax.ShapeDtypeStruct((tout, m, 256), jnp.float32),
        compiler_params=pltpu.CompilerParams(
            dimension_semantics=("parallel", "parallel", "arbitrary")),
    )(x_t, w_t)


def _colmax(a):
    m = a.shape[0]

    def body(a_ref, o_ref):
        @pl.when(pl.program_id(0) == 0)
        def _():
            o_ref[...] = jnp.full_like(o_ref, -jnp.inf)
        mx = jnp.max(a_ref[...], axis=0, keepdims=True)
        o_ref[...] = jnp.maximum(o_ref[...], jnp.broadcast_to(mx, o_ref.shape))

    return pl.pallas_call(
        body,
        grid=(m // MB,),
        in_specs=[pl.BlockSpec((MB, 256), lambda i: (i, 0))],
        out_specs=pl.BlockSpec((8, 256), lambda i: (0, 0)),
        out_shape=jax.ShapeDtypeStruct((8, 256), jnp.float32),
        compiler_params=pltpu.CompilerParams(
            dimension_semantics=("arbitrary",)),
    )(a)


def _den_sum(pden):
    heads = pden.shape[0]

    def body(p_ref, o_ref):
        o_ref[...] = jnp.sum(p_ref[...], axis=1)

    return pl.pallas_call(
        body,
        out_shape=jax.ShapeDtypeStruct((heads, NP), jnp.float32),
    )(pden)


def _finish(acc2, bias_t):
    t = acc2.shape[0]

    def body(a_ref, b_ref, o_ref):
        s = a_ref[0]
        b = jnp.broadcast_to(b_ref[0][0:1, :], s.shape)
        o_ref[...] = jnp.maximum(s + b, 0.0)[None]

    return pl.pallas_call(
        body,
        grid=(N // MB, t),
        in_specs=[
            pl.BlockSpec((1, MB, 256), lambda i, j: (j, i, 0)),
            pl.BlockSpec((1, 8, 256), lambda i, j: (j, 0, 0)),
        ],
        out_specs=pl.BlockSpec((1, MB, 256), lambda i, j: (j, i, 0)),
        out_shape=jax.ShapeDtypeStruct((t, N, 256), jnp.float32),
        compiler_params=pltpu.CompilerParams(
            dimension_semantics=("parallel", "parallel")),
    )(acc2, bias_t)


def _pool_gru(h3_t, p, w_ih1t, b_ih1, b_hh1, w_ih2t, b_ih2, b_hh2, wot, bo):
    def body(h_ref, p_ref, wi1_ref, bi1_ref, bh1_ref, wi2_ref, bi2_ref,
             bh2_ref, wo_ref, bo_ref, o_ref):
        pm = p_ref[...]
        dn = (((0,), (0,)), ((), ()))
        parts = [lax.dot_general(pm, h_ref[tt], dn,
                                 preferred_element_type=jnp.float32)
                 for tt in range(2)]
        ge = jnp.concatenate(parts, axis=1)
        cnt = lax.dot_general(pm, jnp.ones((N, 8), jnp.float32), dn,
                              preferred_element_type=jnp.float32)[:, 0:1]
        ge = ge / jnp.maximum(cnt, 1.0)

        gi1 = jnp.dot(ge, wi1_ref[...], preferred_element_type=jnp.float32)
        gi1 = gi1 + jnp.broadcast_to(bi1_ref[...], gi1.shape)
        bh1 = jnp.broadcast_to(bh1_ref[...], gi1.shape)
        r1 = jax.nn.sigmoid(gi1[:, 0:256] + bh1[:, 0:256])
        z1 = jax.nn.sigmoid(gi1[:, 256:512] + bh1[:, 256:512])
        n1 = jnp.tanh(gi1[:, 512:768] + r1 * bh1[:, 512:768])
        h1 = (1.0 - z1) * n1

        gi2 = jnp.dot(h1, wi2_ref[...], preferred_element_type=jnp.float32)
        gi2 = gi2 + jnp.broadcast_to(bi2_ref[...], gi2.shape)
        bh2 = jnp.broadcast_to(bh2_ref[...], gi2.shape)
        r2 = jax.nn.sigmoid(gi2[:, 0:256] + bh2[:, 0:256])
        z2 = jax.nn.sigmoid(gi2[:, 256:512] + bh2[:, 256:512])
        n2 = jnp.tanh(gi2[:, 512:768] + r2 * bh2[:, 512:768])
        h2 = (1.0 - z2) * n2

        out = jnp.dot(h2, wo_ref[...], preferred_element_type=jnp.float32)
        o_ref[...] = out + jnp.broadcast_to(bo_ref[...], out.shape)

    return pl.pallas_call(
        body,
        out_shape=jax.ShapeDtypeStruct((G, 512), jnp.float32),
    )(h3_t, p, w_ih1t, b_ih1, b_hh1, w_ih2t, b_ih2, b_hh2, wot, bo)


def _sc_mesh():
    return plsc.VectorSubcoreMesh(core_axis_name="c", subcore_axis_name="s")


_SC_PARAMS = pltpu.CompilerParams(needs_layout_passes=False,
                                  use_tc_tiling_on_sc=False)


def _edge_weights(heads, as_t, ad_t, ei4, cvec):
    @functools.partial(
        pl.kernel,
        out_type=(jax.ShapeDtypeStruct((heads, NW, CHN, CHK), jnp.float32),
                  jax.ShapeDtypeStruct((heads, NW, NP), jnp.float32)),
        mesh=_sc_mesh(),
        compiler_params=_SC_PARAMS,
        scratch_types=[
            pltpu.VMEM((CHN, CHK), jnp.int32),
            pltpu.VMEM((CHN, CHK), jnp.int32),
            pltpu.VMEM((NP,), jnp.float32),
            pltpu.VMEM((NP,), jnp.float32),
            pltpu.VMEM((NP,), jnp.float32),
            pltpu.VMEM((CHN, CHK), jnp.float32),
            pltpu.VMEM((16,), jnp.float32),
        ],
    )
    def ek(as_hbm, ad_hbm, ei_hbm, c_hbm, w_out, pden_out,
           src_v, dst_v, as_v, ad_v, den_v, w_v, c_v):
        cc = lax.axis_index("c")
        ss = lax.axis_index("s")
        wid = cc * NS + ss
        pltpu.sync_copy(ei_hbm.at[0, wid], src_v)
        pltpu.sync_copy(ei_hbm.at[1, wid], dst_v)
        pltpu.sync_copy(c_hbm, c_v)
        cv = c_v[...]

        def head_body(h, _):
            pltpu.sync_copy(as_hbm.at[h], as_v)
            pltpu.sync_copy(ad_hbm.at[h], ad_v)

            def zero(i, _):
                den_v[pl.ds(i * 16, 16)] = jnp.zeros((16,), jnp.float32)
                return 0
            lax.fori_loop(0, NP // 16, zero, 0)

            def chunk(j, _):
                for q in range(CHK // 16):
                    s16 = src_v[j, pl.ds(q * 16, 16)]
                    d16 = dst_v[j, pl.ds(q * 16, 16)]
                    av = plsc.load_gather(as_v, [s16])
                    bv = plsc.load_gather(ad_v, [d16])
                    e = av + bv
                    e = jnp.where(e > 0, e, 0.2 * e)
                    wv = jnp.exp(e - cv)
                    w_v[j, pl.ds(q * 16, 16)] = wv
                    plsc.addupdate_scatter(den_v, [d16], wv)
                return 0
            lax.fori_loop(0, CHN, chunk, 0)

            pltpu.sync_copy(w_v, w_out.at[h, wid])
            pltpu.sync_copy(den_v, pden_out.at[h, wid])
            return 0
        lax.fori_loop(0, heads, head_body, 0)

    return ek(as_t, ad_t, ei4, cvec)


def _alphaize(heads, w4, den, ei4):
    @functools.partial(
        pl.kernel,
        out_type=jax.ShapeDtypeStruct((heads, NW, CHN, CHK), jnp.float32),
        mesh=_sc_mesh(),
        compiler_params=_SC_PARAMS,
        scratch_types=[
            pltpu.VMEM((CHN, CHK), jnp.int32),
            pltpu.VMEM((CHN, CHK), jnp.float32),
            pltpu.VMEM((NP,), jnp.float32),
        ],
    )
    def alk(w_hbm, den_hbm, ei_hbm, a_out, dst_v, w_v, den_v):
        cc = lax.axis_index("c")
        ss = lax.axis_index("s")
        wid = cc * NS + ss
        pltpu.sync_copy(ei_hbm.at[1, wid], dst_v)

        def head_body(h, _):
            pltpu.sync_copy(den_hbm.at[h], den_v)
            pltpu.sync_copy(w_hbm.at[h, wid], w_v)

            def chunk(j, _):
                for q in range(CHK // 16):
                    d16 = dst_v[j, pl.ds(q * 16, 16)]
                    dn = plsc.load_gather(den_v, [d16])
                    w_v[j, pl.ds(q * 16, 16)] = (
                        w_v[j, pl.ds(q * 16, 16)] / (dn + 1e-16))
                return 0
            lax.fori_loop(0, CHN, chunk, 0)

            pltpu.sync_copy(w_v, a_out.at[h, wid])
            return 0
        lax.fori_loop(0, heads, head_body, 0)

    return alk(w4, den, ei4)


def _aggregate(dim, tiles, h2d, alpha4, ei4, zer):
    @functools.partial(
        pl.kernel,
        out_type=jax.ShapeDtypeStruct((tiles, NC, ACCR, 256), jnp.float32),
        mesh=_sc_mesh(),
        compiler_params=_SC_PARAMS,
        scratch_types=[
            pltpu.VMEM((CHN, CHK), jnp.int32),
            pltpu.VMEM((CHN, CHK), jnp.int32),
            pltpu.VMEM((2, CHN, CHK), jnp.float32),
            pltpu.VMEM((CHK, 256), jnp.float32),
            pltpu.VMEM((CHK, 256), jnp.float32),
            pltpu.VMEM_SHARED((ACCR, 256), jnp.float32),
            pltpu.SemaphoreType.DMA,
            pltpu.SemaphoreType.DMA,
        ],
    )
    def ak(h_hbm, a_hbm, ei_hbm, z_hbm, acc_out,
           src_v, dst_v, a_v, rows_0, rows_1, acc_sp, sem_0, sem_1):
        rows_bufs = (rows_0, rows_1)
        sems = (sem_0, sem_1)
        cc = lax.axis_index("c")
        ss = lax.axis_index("s")
        wid = cc * NS + ss
        pltpu.sync_copy(ei_hbm.at[0, wid], src_v)
        pltpu.sync_copy(ei_hbm.at[1, wid], dst_v)

        base = cc * NH

        lane = lax.iota(jnp.int32, 16)

        def localize(j, _):
            for q in range(CHK // 16):
                d16 = dst_v[j, pl.ds(q * 16, 16)]
                loc = jnp.minimum(jnp.maximum(d16 - base, 0), NH)
                sp = j * CHK + q * 16 + lane
                sp = jnp.where(sp >= NH, sp - NH, sp)
                d16 = jnp.where(d16 == N, sp, loc)
                dst_v[j, pl.ds(q * 16, 16)] = d16
            return 0
        lax.fori_loop(0, CHN, localize, 0)

        def scale_scatter(j, rows):
            @plsc.parallel_loop(0, CHK, unroll=8)
            def _(i):
                wv0 = plsc.load_gather(
                    a_v, [jnp.full((16,), 0, jnp.int32),
                          jnp.full((16,), j, jnp.int32),
                          jnp.full((16,), i, jnp.int32)])
                wv1 = plsc.load_gather(
                    a_v, [jnp.full((16,), 1, jnp.int32),
                          jnp.full((16,), j, jnp.int32),
                          jnp.full((16,), i, jnp.int32)])
                for q in range(8):
                    rows[i, pl.ds(q * 16, 16)] = (
                        rows[i, pl.ds(q * 16, 16)] * wv0)
                for q in range(8, 16):
                    rows[i, pl.ds(q * 16, 16)] = (
                        rows[i, pl.ds(q * 16, 16)] * wv1)

            pltpu.sync_copy(rows, acc_sp.at[dst_v.at[j]], add=True)
            for q in range(CHK // 16):
                src_v[j, pl.ds(q * 16, 16)] = (
                    src_v[j, pl.ds(q * 16, 16)] + N)

        def tile_body(t, _):
            hh0 = (t * 256) // dim
            hh1 = (t * 256 + 128) // dim
            pltpu.sync_copy(a_hbm.at[hh0, wid], a_v.at[0])
            pltpu.sync_copy(a_hbm.at[hh1, wid], a_v.at[1])

            pltpu.sync_copy(z_hbm, acc_sp.at[pl.ds(ss * FLR, FLR)])
            plsc.subcore_barrier()

            for b in range(2):
                pltpu.async_copy(h_hbm.at[src_v.at[b]],
                                 rows_bufs[b], sems[b])

            def duo(jj, _):
                j0 = jj * 2
                for b in range(2):
                    j = j0 + b
                    pltpu.make_async_copy(
                        h_hbm.at[src_v.at[j]], rows_bufs[b], sems[b]).wait()
                    scale_scatter(j, rows_bufs[b])

                    @pl.when(j + 2 < CHN)
                    def _():
                        pltpu.async_copy(h_hbm.at[src_v.at[j + 2]],
                                         rows_bufs[b], sems[b])
                return 0
            lax.fori_loop(0, CHN // 2, duo, 0)
            plsc.subcore_barrier()

            sl = pl.ds(ss * FLR, FLR)
            pltpu.sync_copy(acc_sp.at[sl], acc_out.at[t, cc, sl])
            plsc.subcore_barrier()
            return 0
        lax.fori_loop(0, tiles, tile_body, 0)

    return ak(h2d, alpha4, ei4, zer)


def _gat_layer(x_t, w_t, a_src, a_dst, bias, heads, dim, ei4, zer):
    hp_t = _mm_tiled(x_t, w_t)
    t = hp_t.shape[0]
    k = heads * dim

    eye = jnp.eye(heads, dtype=jnp.float32)
    a_s = (eye[:, None, :] * a_src[:, :, None]).reshape(k, heads)
    a_d = (eye[:, None, :] * a_dst[:, :, None]).reshape(k, heads)
    a_cat = jnp.concatenate([a_s, a_d], axis=1)
    a_cat = jnp.pad(a_cat, ((0, 0), (0, 256 - 2 * heads)))
    a_cat = a_cat.reshape(k // 256, 256, 256)

    al = _mm_tiled(hp_t, a_cat)[0]
    m8 = _colmax(al)
    cb = jnp.maximum(
        jnp.max(m8[:, :heads]) + jnp.max(m8[:, heads:2 * heads]), 0.0)
    cvec = jnp.full((16,), cb, jnp.float32)

    al_tr = al.T
    as_t = jnp.pad(al_tr[:heads], ((0, 0), (0, NP - N)))
    ad_t = jnp.pad(al_tr[heads:2 * heads], ((0, 0), (0, NP - N)),
                   constant_values=-1e30)

    w_e, pden = _edge_weights(heads, as_t, ad_t, ei4, cvec)
    den = _den_sum(pden)
    alpha4 = _alphaize(heads, w_e, den, ei4)
    acc = _aggregate(dim, t, hp_t.reshape(t * N, 256), alpha4, ei4, zer)
    acc2 = acc[:, :, :NH, :].reshape(t, N, 256)
    bias_t = jnp.broadcast_to(bias.reshape(t, 1, 256), (t, 8, 256))
    return _finish(acc2, bias_t)


def kernel(x, edge_index, batch_idx, W1, a_src1, a_dst1, b1, W2, a_src2,
           a_dst2, b2, W3, a_src3, a_dst3, b3, W_ih1, W_hh1, b_ih1, b_hh1,
           W_ih2, W_hh2, b_ih2, b_hh2, Wo, bo):
    ei = edge_index.astype(jnp.int32)
    src, dst = ei[0], ei[1]
    half = (dst >= NH).astype(jnp.int32)
    order = jnp.argsort(half * E + jnp.arange(E, dtype=jnp.int32))
    p0 = E - jnp.sum(half)
    k = jnp.arange(EP2, dtype=jnp.int32)
    idx = jnp.where(k < CAP, k, p0 + (k - CAP))
    valid = jnp.where(k < CAP, k < p0, idx < E)
    idx = jnp.clip(idx, 0, E - 1)
    g = order[idx]
    bsrc = jnp.where(valid, src[g], 0)
    bdst = jnp.where(valid, dst[g], N)
    ei4 = jnp.stack([bsrc, bdst]).reshape(2, NW, CHN, CHK)
    zer = jnp.zeros((FLR, 256), jnp.float32)

    x_t = jnp.pad(x, ((0, 0), (0, 256 - 47)))[None]
    w1_t = jnp.pad(W1, ((0, 256 - 47), (0, 0)))[None]
    h1_t = _gat_layer(x_t, w1_t, a_src1, a_dst1, b1, 8, 128, ei4, zer)
    h2_t = _gat_layer(h1_t, W2.reshape(4, 256, 2048), a_src2, a_dst2, b2,
                      8, 256, ei4, zer)
    h3_t = _gat_layer(h2_t, W3.reshape(8, 256, 512), a_src3, a_dst3, b3,
                      1, 512, ei4, zer)

    p = (batch_idx[:, None] == jnp.arange(G, dtype=batch_idx.dtype)[None, :])
    p = p.astype(jnp.float32)
    return _pool_gru(h3_t, p, W_ih1.T, b_ih1[None], b_hh1[None], W_ih2.T,
                     b_ih2[None], b_hh2[None], Wo.T, bo[None])
